# Initial kernel scaffold; baseline (speedup 1.0000x reference)
import jax
import jax.numpy as jnp
from jax.experimental import pallas as pl


def kernel(inputs, table):
    raise NotImplementedError("write your pallas kernel here")


# speedup vs baseline: 3.4713x; 3.4713x over previous
import functools

import jax
import jax.numpy as jnp
from jax import lax
from jax.experimental import pallas as pl
from jax.experimental.pallas import tpu as pltpu
from jax.experimental.pallas import tpu_sc as plsc

EMBED = 64
NC = 2
NS = 16
NW = NC * NS
CHUNK = 128


@functools.lru_cache(maxsize=None)
def _build(n_chunks: int, total: int):
    mesh = plsc.VectorSubcoreMesh(core_axis_name="c", subcore_axis_name="s")
    per_w = n_chunks * CHUNK

    @functools.partial(
        pl.kernel,
        mesh=mesh,
        out_type=jax.ShapeDtypeStruct((total, EMBED), jnp.float32),
        scratch_types=[
            pltpu.VMEM((n_chunks, CHUNK), jnp.int32),
            pltpu.VMEM((CHUNK, EMBED), jnp.float32),
            pltpu.SemaphoreType.DMA,
        ],
        compiler_params=pltpu.CompilerParams(use_tc_tiling_on_sc=False),
    )
    def emb(idx_hbm, table_hbm, out_hbm, idx_v, rows_v, sem):
        wid = lax.axis_index("s") * NC + lax.axis_index("c")
        pltpu.sync_copy(idx_hbm.at[wid], idx_v)
        base = wid * per_w

        def body(j, carry):
            pltpu.async_copy(table_hbm.at[idx_v.at[j]], rows_v, sem).wait()
            pltpu.sync_copy(rows_v, out_hbm.at[pl.ds(base + j * CHUNK, CHUNK)])
            return carry

        lax.fori_loop(0, n_chunks, body, 0)

    return emb


def kernel(inputs, table):
    batch, seq = inputs.shape
    total = batch * seq
    n_chunks = total // (NW * CHUNK)
    idx = inputs.reshape(NW, n_chunks, CHUNK).astype(jnp.int32)
    out = _build(n_chunks, total)(idx, table)
    return out.reshape(batch, seq, EMBED)


# speedup vs baseline: 3.6186x; 1.0424x over previous
import functools

import jax
import jax.numpy as jnp
from jax import lax
from jax.experimental import pallas as pl
from jax.experimental.pallas import tpu as pltpu
from jax.experimental.pallas import tpu_sc as plsc

EMBED = 64
NC = 2
NS = 16
NW = NC * NS
CHUNK = 128
NBUF = 4


@functools.lru_cache(maxsize=None)
def _build(n_chunks: int, total: int):
    mesh = plsc.VectorSubcoreMesh(core_axis_name="c", subcore_axis_name="s")
    per_w = n_chunks * CHUNK
    n_groups = n_chunks // NBUF
    assert n_chunks % NBUF == 0 and n_groups >= 2

    @functools.partial(
        pl.kernel,
        mesh=mesh,
        out_type=jax.ShapeDtypeStruct((total, EMBED), jnp.float32),
        scratch_types=[
            pltpu.VMEM((n_chunks, CHUNK), jnp.int32),
            pltpu.VMEM((NBUF, CHUNK, EMBED), jnp.float32),
            pltpu.SemaphoreType.DMA((NBUF,)),
            pltpu.SemaphoreType.DMA((NBUF,)),
        ],
        compiler_params=pltpu.CompilerParams(use_tc_tiling_on_sc=False),
    )
    def emb(idx_hbm, table_hbm, out_hbm, idx_v, rows_v, gsem, osem):
        wid = lax.axis_index("s") * NC + lax.axis_index("c")
        pltpu.sync_copy(idx_hbm.at[wid], idx_v)
        base = wid * per_w

        def fire_gather(k, b):
            pltpu.async_copy(table_hbm.at[idx_v.at[k]], rows_v.at[b],
                             gsem.at[b])

        def wait_gather(b):
            pltpu.make_async_copy(out_hbm.at[pl.ds(base, CHUNK)],
                                  rows_v.at[b], gsem.at[b]).wait()

        def fire_out(k, b):
            pltpu.async_copy(rows_v.at[b],
                             out_hbm.at[pl.ds(base + k * CHUNK, CHUNK)],
                             osem.at[b])

        def wait_out(b):
            pltpu.make_async_copy(rows_v.at[b],
                                  out_hbm.at[pl.ds(base, CHUNK)],
                                  osem.at[b]).wait()

        def step(k, b, prefetch):
            wait_gather(b)
            fire_out(k, b)
            if prefetch:
                bp = (b - 1) % NBUF
                wait_out(bp)
                fire_gather(k - 1 + NBUF, bp)

        for b in range(NBUF):
            fire_gather(b, b)
        for b in range(NBUF):
            step(b, b, prefetch=b > 0)

        def group(g, carry):
            for b in range(NBUF):
                step(g * NBUF + b, b, prefetch=True)
            return carry

        lax.fori_loop(1, n_groups - 1, group, 0)

        for b in range(NBUF):
            step(n_chunks - NBUF + b, b, prefetch=b == 0)
        for b in range(NBUF):
            wait_out(b)

    return emb


def kernel(inputs, table):
    batch, seq = inputs.shape
    total = batch * seq
    n_chunks = total // (NW * CHUNK)
    idx = inputs.reshape(NW, n_chunks, CHUNK).astype(jnp.int32)
    out = _build(n_chunks, total)(idx, table)
    return out.reshape(batch, seq, EMBED)


# speedup vs baseline: 4.5451x; 1.2561x over previous
import functools

import jax
import jax.numpy as jnp
from jax import lax
from jax.experimental import pallas as pl
from jax.experimental.pallas import tpu as pltpu
from jax.experimental.pallas import tpu_sc as plsc

EMBED = 64
NC = 2
NS = 16
NW = NC * NS
CHUNK = 128
NBUF = 4


@functools.lru_cache(maxsize=None)
def _build(n_chunks: int, total: int, vocab: int):
    mesh = plsc.VectorSubcoreMesh(core_axis_name="c", subcore_axis_name="s")
    per_w = n_chunks * CHUNK
    n_groups = n_chunks // NBUF
    assert n_chunks % NBUF == 0 and n_groups >= 2

    @functools.partial(
        pl.kernel,
        mesh=mesh,
        out_type=jax.ShapeDtypeStruct((total, EMBED), jnp.float32),
        scratch_types=[
            pltpu.VMEM((n_chunks, CHUNK), jnp.int32),
            pltpu.VMEM((NBUF, CHUNK, EMBED), jnp.float32),
            pltpu.VMEM_SHARED((vocab, EMBED), jnp.float32),
            pltpu.SemaphoreType.DMA((NBUF,)),
            pltpu.SemaphoreType.DMA((NBUF,)),
        ],
        compiler_params=pltpu.CompilerParams(use_tc_tiling_on_sc=False),
    )
    def emb(idx_hbm, table_hbm, out_hbm, idx_v, rows_v, table_sh, gsem, osem):
        sid = lax.axis_index("s")
        wid = sid * NC + lax.axis_index("c")

        @pl.when(sid == 0)
        def _stage():
            pltpu.sync_copy(table_hbm, table_sh)

        pltpu.sync_copy(idx_hbm.at[wid], idx_v)
        plsc.subcore_barrier()
        base = wid * per_w

        def fire_gather(k, b):
            pltpu.async_copy(table_sh.at[idx_v.at[k]], rows_v.at[b],
                             gsem.at[b])

        def wait_gather(b):
            pltpu.make_async_copy(out_hbm.at[pl.ds(base, CHUNK)],
                                  rows_v.at[b], gsem.at[b]).wait()

        def fire_out(k, b):
            pltpu.async_copy(rows_v.at[b],
                             out_hbm.at[pl.ds(base + k * CHUNK, CHUNK)],
                             osem.at[b])

        def wait_out(b):
            pltpu.make_async_copy(rows_v.at[b],
                                  out_hbm.at[pl.ds(base, CHUNK)],
                                  osem.at[b]).wait()

        def step(k, b, prefetch):
            wait_gather(b)
            fire_out(k, b)
            if prefetch:
                bp = (b - 1) % NBUF
                wait_out(bp)
                fire_gather(k - 1 + NBUF, bp)

        for b in range(NBUF):
            fire_gather(b, b)
        for b in range(NBUF):
            step(b, b, prefetch=b > 0)

        def group(g, carry):
            for b in range(NBUF):
                step(g * NBUF + b, b, prefetch=True)
            return carry

        lax.fori_loop(1, n_groups - 1, group, 0)

        for b in range(NBUF):
            step(n_chunks - NBUF + b, b, prefetch=b == 0)
        for b in range(NBUF):
            wait_out(b)

    return emb


def kernel(inputs, table):
    batch, seq = inputs.shape
    total = batch * seq
    n_chunks = total // (NW * CHUNK)
    idx = inputs.reshape(NW, n_chunks, CHUNK).astype(jnp.int32)
    out = _build(n_chunks, total, table.shape[0])(idx, table)
    return out.reshape(batch, seq, EMBED)
